# Initial kernel scaffold; baseline (speedup 1.0000x reference)
import jax
import jax.numpy as jnp
from jax.experimental import pallas as pl


def kernel(x, prompt, w_g, w_n, fc1_w, fc1_b, fc2_w, fc2_b):
    raise NotImplementedError("write your pallas kernel here")


# speedup vs baseline: 4.2118x; 4.2118x over previous
import functools

import jax
import jax.numpy as jnp
import numpy as np
from jax.experimental import pallas as pl
from jax.experimental.pallas import tpu as pltpu

_EPS_FLOOR = float(np.finfo(float).eps)


def _gelu_exact(v):
    return v * 0.5 * (1.0 + jax.lax.erf(v * np.float32(1.0 / np.sqrt(2.0))))


def _dense_moe_kernel(
    x_p_ref, w_g_ref, fc1_w_ref, fc1_b_ref, fc2_w_ref, fc2_b_ref,
    y_ref, loss_ref,
    w_scr, stat_scr,
    *, blk, n_tok_blocks, n_experts,
):
    t = pl.program_id(0)
    e = pl.program_id(1)

    @pl.when(e == 0)
    def _router():
        x_p = x_p_ref[...].astype(jnp.bfloat16)
        logits = jax.lax.dot_general(
            x_p, w_g_ref[...].astype(jnp.bfloat16), (((1,), (0,)), ((), ())),
            preferred_element_type=jnp.float32,
        )
        col = jax.lax.broadcasted_iota(jnp.int32, logits.shape, 1)
        i1 = jnp.argmax(logits, axis=1, keepdims=True)
        v1 = jnp.max(logits, axis=1, keepdims=True)
        masked = jnp.where(col == i1, -jnp.inf, logits)
        i2 = jnp.argmax(masked, axis=1, keepdims=True)
        v2 = jnp.max(masked, axis=1, keepdims=True)
        e2 = jnp.exp(v2 - v1)
        g1 = 1.0 / (1.0 + e2)
        g2 = e2 / (1.0 + e2)
        w_blk = jnp.where(col == i1, g1, jnp.where(col == i2, g2, 0.0))
        w_scr[...] = w_blk

        @pl.when(t == 0)
        def _init_stats():
            stat_scr[...] = jnp.zeros_like(stat_scr)

        stat_scr[0, :] += jnp.sum(w_blk, axis=0)
        stat_scr[1, :] += jnp.sum((w_blk > 0).astype(jnp.float32), axis=0)

    x = x_p_ref[:, : fc1_w_ref.shape[2]].astype(jnp.bfloat16)
    h1 = jax.lax.dot_general(
        x, fc1_w_ref[0].astype(jnp.bfloat16), (((1,), (1,)), ((), ())),
        preferred_element_type=jnp.float32,
    ) + fc1_b_ref[0]
    h1 = _gelu_exact(h1).astype(jnp.bfloat16)
    o = jax.lax.dot_general(
        h1, fc2_w_ref[0].astype(jnp.bfloat16), (((1,), (1,)), ((), ())),
        preferred_element_type=jnp.float32,
    ) + fc2_b_ref[0]
    w_all = w_scr[...]
    col_e = jax.lax.broadcasted_iota(jnp.int32, w_all.shape, 1)
    w_e = jnp.sum(jnp.where(col_e == e, w_all, 0.0), axis=1, keepdims=True)
    contrib = jnp.exp(o) * w_e

    @pl.when(e == 0)
    def _init_acc():
        y_ref[...] = contrib

    @pl.when(e > 0)
    def _acc():
        y_ref[...] += contrib

    @pl.when(e == n_experts - 1)
    def _finalize():
        acc = y_ref[...]
        y_ref[...] = jnp.log(jnp.where(acc == 0.0, _EPS_FLOOR, acc))

    @pl.when((t == n_tok_blocks - 1) & (e == n_experts - 1))
    def _loss():
        def balance(v):
            m = jnp.mean(v)
            var = jnp.sum((v - m) ** 2) / (v.shape[-1] - 1)
            return var / (m * m + 1e-10)

        loss_ref[0, 0] = balance(stat_scr[0, :]) + balance(stat_scr[1, :])


@functools.partial(jax.jit, static_argnames=())
def kernel(x, prompt, w_g, w_n, fc1_w, fc1_b, fc2_w, fc2_b):
    del w_n
    B, C, H, W = x.shape
    E = w_g.shape[1]
    hid = fc1_w.shape[1]
    T = B * H * W
    BLK = 512
    n_tok_blocks = T // BLK

    xt = jnp.transpose(x, (0, 2, 3, 1)).reshape(T, C)
    pt = jnp.broadcast_to(prompt[:, None, None, :], (B, H, W, C)).reshape(T, C)
    x_p = jnp.concatenate([xt, pt], axis=1)

    grid = (n_tok_blocks, E)
    y_flat, loss = pl.pallas_call(
        functools.partial(
            _dense_moe_kernel, blk=BLK, n_tok_blocks=n_tok_blocks, n_experts=E
        ),
        grid=grid,
        in_specs=[
            pl.BlockSpec((BLK, 2 * C), lambda t, e: (t, 0)),
            pl.BlockSpec((2 * C, E), lambda t, e: (0, 0)),
            pl.BlockSpec((1, hid, C), lambda t, e: (e, 0, 0)),
            pl.BlockSpec((1, 1, hid), lambda t, e: (e, 0, 0)),
            pl.BlockSpec((1, C, hid), lambda t, e: (e, 0, 0)),
            pl.BlockSpec((1, 1, C), lambda t, e: (e, 0, 0)),
        ],
        out_specs=[
            pl.BlockSpec((BLK, C), lambda t, e: (t, 0)),
            pl.BlockSpec(memory_space=pltpu.SMEM),
        ],
        out_shape=[
            jax.ShapeDtypeStruct((T, C), jnp.float32),
            jax.ShapeDtypeStruct((1, 1), jnp.float32),
        ],
        scratch_shapes=[
            pltpu.VMEM((BLK, E), jnp.float32),
            pltpu.VMEM((2, E), jnp.float32),
        ],
    )(x_p, w_g, fc1_w, fc1_b.reshape(E, 1, hid), fc2_w, fc2_b.reshape(E, 1, C))

    y = y_flat.reshape(B, H, W, C).transpose(0, 3, 1, 2)
    return y, loss[0, 0]
